# baseline (speedup 1.0000x reference)
import functools
import jax
import jax.numpy as jnp
from jax import lax
from jax.experimental import pallas as pl
from jax.experimental.pallas import tpu as pltpu
from jax.experimental.pallas import tpu_sc as plsc

_NC = 2
_NS = 16
_L = 16


def _sc_body(vpad, out, x_ref, y_ref, *, n, d, nb, r_pairs):
    half = n // 2
    ngrp = d // _L
    ntask = nb * ngrp
    nw = _NC * _NS
    per_w = ntask // nw
    nchunk = half // r_pairs

    wid = lax.axis_index("s") * _NC + lax.axis_index("c")
    lane = lax.iota(jnp.int32, _L)

    for t in range(per_w):
        task = wid * per_w + t
        b = task // ngrp
        g = task % ngrp
        col0 = g * _L
        delta0 = (1 - jnp.minimum(lane, 1)) * (1 - jnp.minimum(g, 1))
        base_a = 2 * _L - 2 * lane - 2 * delta0

        for m in range(nchunk):
            a = m * r_pairs
            w0 = (a - col0 - (_L - 1) + half) % half
            pltpu.sync_copy(
                vpad.at[b, pl.ds(2 * w0, 2 * (r_pairs + _L)),
                        pl.ds(col0, _L)],
                x_ref)

            def body(kr, row_a):
                va = plsc.load_gather(x_ref, [row_a, lane])
                vb = plsc.load_gather(x_ref, [row_a + 1, lane])
                y_ref[2 * kr] = jnp.minimum(va, vb)
                y_ref[2 * kr + 1] = jnp.maximum(va, vb)
                return row_a + 2

            lax.fori_loop(0, r_pairs, body, base_a, unroll=4)

            pltpu.sync_copy(
                y_ref,
                out.at[b, pl.ds(2 * a, 2 * r_pairs), pl.ds(col0, _L)])


def kernel(v):
    nb, n, d = v.shape
    half = n // 2
    r_pairs = 512
    pad = 2 * (r_pairs + _L)
    vpad = jnp.concatenate([v, v[:, :pad, :]], axis=1)

    mesh = plsc.VectorSubcoreMesh(core_axis_name="c", subcore_axis_name="s",
                                  num_cores=_NC, num_subcores=_NS)
    body = functools.partial(_sc_body, n=n, d=d, nb=nb, r_pairs=r_pairs)
    f = pl.kernel(
        body,
        out_type=jax.ShapeDtypeStruct((nb, n, d), v.dtype),
        mesh=mesh,
        compiler_params=pltpu.CompilerParams(use_tc_tiling_on_sc=False,
                                             needs_layout_passes=False),
        scratch_types=[
            pltpu.VMEM((2 * (r_pairs + _L), _L), jnp.float32),
            pltpu.VMEM((2 * r_pairs, _L), jnp.float32),
        ],
    )
    return f(vpad)


# speedup vs baseline: 3.0041x; 3.0041x over previous
import functools
import jax
import jax.numpy as jnp
from jax.experimental import pallas as pl


def _swd_block(v_ref, o_ref, *, lanes, n_rows):
    c = pl.program_id(1)
    x = v_ref[0]
    xm1 = jnp.roll(x, -1, axis=0)
    xm2 = jnp.roll(x, -2, axis=0)
    xm3 = jnp.roll(x, -3, axis=0)
    row = jax.lax.broadcasted_iota(jnp.int32, (n_rows, 1), 0)
    even = (row & 1) == 0
    z = jnp.where(even, jnp.minimum(xm2, xm3), jnp.maximum(xm2, xm1))

    lane = jax.lax.broadcasted_iota(jnp.int32, (1, lanes), 1)
    nbits = max(1, (lanes - 1).bit_length())
    for k in range(0, nbits, 2):
        a = 2 << k
        bit_a = ((lane >> k) & 1) == 1
        if k + 1 < nbits:
            b = 4 << k
            bit_b = ((lane >> (k + 1)) & 1) == 1
            t1 = jnp.where(bit_a, jnp.roll(z, a, axis=0), z)
            t2 = jnp.where(bit_a, jnp.roll(z, a + b, axis=0),
                           jnp.roll(z, b, axis=0))
            z = jnp.where(bit_b, t2, t1)
        else:
            z = jnp.where(bit_a, jnp.roll(z, a, axis=0), z)

    z = jax.lax.cond(
        c == 0,
        lambda t: jnp.where(lane == 0, jnp.roll(t, 2, axis=0), t),
        lambda t: t,
        z)

    ch = 2 * lanes
    base = 2 * lanes * c
    for p in range(0, n_rows, ch):
        row0 = (p + base) % n_rows
        o_ref[0, pl.ds(row0, ch), :] = z[p:p + ch, :]


def kernel(v, interpret=False):
    b, n, d = v.shape
    lanes = min(128, d)
    grid = (b, d // lanes)
    body = functools.partial(_swd_block, lanes=lanes, n_rows=n)
    return pl.pallas_call(
        body,
        grid=grid,
        in_specs=[pl.BlockSpec((1, n, lanes), lambda i, j: (i, 0, j))],
        out_specs=pl.BlockSpec((1, n, lanes), lambda i, j: (i, 0, j)),
        out_shape=jax.ShapeDtypeStruct((b, n, d), v.dtype),
        interpret=interpret,
    )(v)
